# Initial kernel scaffold; baseline (speedup 1.0000x reference)
import jax
import jax.numpy as jnp
from jax.experimental import pallas as pl


def kernel(idiom_ids, table, gamma, beta):
    raise NotImplementedError("write your pallas kernel here")


# speedup vs baseline: 1.5273x; 1.5273x over previous
import functools

import jax
import jax.numpy as jnp
from jax import lax
from jax.experimental import pallas as pl
from jax.experimental.pallas import tpu as pltpu
from jax.experimental.pallas import tpu_sc as plsc

_HIDDEN = 64
_EPS = 1e-12
_NC = 2
_NS = 16
_NW = _NC * _NS


def _ln_body(rpw, chunk, idx_hbm, table_hbm, gamma_hbm, beta_hbm, out_hbm,
             idx_v, rows_v, gb_v, sem):
    wid = lax.axis_index("s") * _NC + lax.axis_index("c")
    base = wid * rpw
    nchunk = rpw // chunk

    pltpu.sync_copy(gamma_hbm, gb_v.at[0])
    pltpu.sync_copy(beta_hbm, gb_v.at[1])
    g = [gb_v[0, pl.ds(16 * h, 16)] for h in range(4)]
    b = [gb_v[1, pl.ds(16 * h, 16)] for h in range(4)]
    lanes = lax.iota(jnp.int32, 16)
    perms = [lax.bitwise_xor(lanes, jnp.int32(1 << p)) for p in range(4)]

    dnums = lax.GatherDimensionNumbers(
        offset_dims=(), collapsed_slice_dims=(0,), start_index_map=(0,))

    def _allsum(x):
        for p in perms:
            x = x + lax.gather(
                x, p.reshape(16, 1), dnums, (1,),
                indices_are_sorted=False, unique_indices=True,
                mode=lax.GatherScatterMode.PROMISE_IN_BOUNDS)
        return x

    def chunk_body(c, _):
        start = base + c * chunk
        pltpu.sync_copy(idx_hbm.at[pl.ds(start, chunk)], idx_v)
        pltpu.async_copy(table_hbm.at[idx_v], rows_v, sem).wait()

        def row_body(r, _):
            x = [rows_v[r, pl.ds(16 * h, 16)] for h in range(4)]
            s = (x[0] + x[1]) + (x[2] + x[3])
            sq = (x[0] * x[0] + x[1] * x[1]) + (x[2] * x[2] + x[3] * x[3])
            mean = _allsum(s) * (1.0 / 64.0)
            ex2 = _allsum(sq) * (1.0 / 64.0)
            v = ex2 - mean * mean + _EPS
            i = lax.bitcast_convert_type(v, jnp.int32)
            i = jnp.int32(0x5F3759DF) - lax.shift_right_logical(i, 1)
            y = lax.bitcast_convert_type(i, jnp.float32)
            hv = 0.5 * v
            y = y * (1.5 - hv * y * y)
            y = y * (1.5 - hv * y * y)
            y = y * (1.5 - hv * y * y)
            for h in range(4):
                rows_v[r, pl.ds(16 * h, 16)] = (x[h] - mean) * y * g[h] + b[h]
            return 0

        lax.fori_loop(0, chunk, row_body, 0, unroll=2)
        pltpu.sync_copy(rows_v, out_hbm.at[pl.ds(start, chunk)])
        return 0

    lax.fori_loop(0, nchunk, chunk_body, 0)


def _make_call(rows, chunk):
    rpw = rows // _NW
    mesh = plsc.VectorSubcoreMesh(core_axis_name="c", subcore_axis_name="s")
    return pl.kernel(
        functools.partial(_ln_body, rpw, chunk),
        out_type=jax.ShapeDtypeStruct((rows, _HIDDEN), jnp.float32),
        mesh=mesh,
        scratch_types=[
            pltpu.VMEM((chunk,), jnp.int32),
            pltpu.VMEM((chunk, _HIDDEN), jnp.float32),
            pltpu.VMEM((2, _HIDDEN), jnp.float32),
            pltpu.SemaphoreType.DMA,
        ],
        compiler_params=pltpu.CompilerParams(use_tc_tiling_on_sc=False),
    )


@jax.jit
def kernel(idiom_ids, table, gamma, beta):
    bsz, seq = idiom_ids.shape
    rows = bsz * seq
    idx = idiom_ids.reshape(rows).astype(jnp.int32)
    out = _make_call(rows, 1024)(idx, table, gamma, beta)
    return out.reshape(bsz, seq, _HIDDEN)
